# baseline (speedup 1.0000x reference)
import functools

import jax
import jax.numpy as jnp
from jax import lax
from jax.experimental import pallas as pl
from jax.experimental.pallas import tpu as pltpu
from jax.experimental.pallas import tpu_sc as plsc

N_NODE = 10000
N_EDGE = 2000
NNZ = 320000
D = 128
W = 144
ONES_COL = 128
EDGE_PAD = 2048
NODE_PAD = 10240
BLK = 1024
FBLK = 1024


def _mm_body(x_ref, w_ref, o_ref):
    acc = jnp.dot(x_ref[...], w_ref[...], preferred_element_type=jnp.float32)
    col = lax.broadcasted_iota(jnp.int32, (BLK, W), 1)
    o_ref[...] = acc + jnp.where(col == ONES_COL, 1.0, 0.0)


def _mm(x, wp):
    return pl.pallas_call(
        _mm_body,
        grid=(NODE_PAD // BLK,),
        in_specs=[
            pl.BlockSpec((BLK, D), lambda i: (i, 0)),
            pl.BlockSpec((D, W), lambda i: (0, 0)),
        ],
        out_specs=pl.BlockSpec((BLK, W), lambda i: (i, 0)),
        out_shape=jax.ShapeDtypeStruct((NODE_PAD, W), jnp.float32),
    )(x, wp)


def _merge_body(a_ref, b_ref, o_ref):
    t = a_ref[...] + b_ref[...]
    cnt = jnp.maximum(t[:, ONES_COL:ONES_COL + 1], 1.0)
    o_ref[...] = t / cnt


def _merge(ep):
    return pl.pallas_call(
        _merge_body,
        grid=(1,),
        in_specs=[
            pl.BlockSpec((EDGE_PAD, W), lambda i: (0, 0)),
            pl.BlockSpec((EDGE_PAD, W), lambda i: (1, 0)),
        ],
        out_specs=pl.BlockSpec((EDGE_PAD, W), lambda i: (0, 0)),
        out_shape=jax.ShapeDtypeStruct((EDGE_PAD, W), jnp.float32),
    )(ep, ep)


def _final_body(a_ref, b_ref, bias_ref, o_ref):
    t = a_ref[...] + b_ref[...]
    cnt = jnp.maximum(t[:, ONES_COL:ONES_COL + 1], 1.0)
    o_ref[...] = t[:, :D] / cnt + bias_ref[...]


def _final(vp, bias2d):
    nb = NODE_PAD // FBLK
    return pl.pallas_call(
        _final_body,
        grid=(nb,),
        in_specs=[
            pl.BlockSpec((FBLK, W), lambda i: (i, 0)),
            pl.BlockSpec((FBLK, W), lambda i: (i + nb, 0)),
            pl.BlockSpec((1, D), lambda i: (0, 0)),
        ],
        out_specs=pl.BlockSpec((FBLK, D), lambda i: (i, 0)),
        out_shape=jax.ShapeDtypeStruct((NODE_PAD, D), jnp.float32),
    )(vp, vp, bias2d)


def _make_sc_phase(n_src, n_dst, B, G):
    rows_per_tile = n_dst // 16
    src_per_tile = n_src // 16
    chunk = NNZ // 32
    nb = chunk // B
    ng = nb // G
    assert nb % G == 0
    mesh = plsc.VectorSubcoreMesh(core_axis_name="c", subcore_axis_name="s")

    @functools.partial(
        pl.kernel,
        mesh=mesh,
        compiler_params=pltpu.CompilerParams(use_tc_tiling_on_sc=False),
        out_type=jax.ShapeDtypeStruct((2 * n_dst, W), jnp.float32),
        scratch_types=[
            pltpu.VMEM((G, B), jnp.int32),
            pltpu.VMEM((2, G, B), jnp.int32),
            pltpu.VMEM_SHARED((n_dst, W), jnp.float32),
            pltpu.VMEM_SHARED((n_src, W), jnp.float32),
        ]
        + [pltpu.VMEM((B, W), jnp.float32) for _ in range(G)]
        + [pltpu.SemaphoreType.DMA for _ in range(2 * G)],
    )
    def phase(src_hbm, gidx_hbm, didx_hbm, z_hbm, out_hbm,
              gibuf, dibuf, acc, srcbuf, *rest):
        rows = rest[:G]
        gsem = rest[G:2 * G]
        ssem = rest[2 * G:]
        c = lax.axis_index("c")
        s = lax.axis_index("s")
        r0 = s * rows_per_tile
        pltpu.sync_copy(z_hbm.at[pl.ds(r0, rows_per_tile)],
                        acc.at[pl.ds(r0, rows_per_tile)])
        p0 = s * src_per_tile
        pltpu.sync_copy(src_hbm.at[pl.ds(p0, src_per_tile)],
                        srcbuf.at[pl.ds(p0, src_per_tile)])
        tb = (c * 16 + s) * nb
        pltpu.sync_copy(gidx_hbm.at[pl.ds(tb, G)], gibuf)
        pltpu.sync_copy(didx_hbm.at[pl.ds(tb, G)], dibuf.at[0])
        plsc.subcore_barrier()

        def group(g, carry):
            p = lax.rem(g, 2)
            gd = [pltpu.async_copy(srcbuf.at[gibuf.at[b]],
                                   rows[b], gsem[b]) for b in range(G)]
            sd = []
            for b in range(G):
                gd[b].wait()
                sd.append(pltpu.async_copy(rows[b], acc.at[dibuf.at[p, b]],
                                           ssem[b], add=True))
            nxt = tb + jnp.minimum((g + 1) * G, nb - G)
            pltpu.sync_copy(gidx_hbm.at[pl.ds(nxt, G)], gibuf)
            pltpu.sync_copy(didx_hbm.at[pl.ds(nxt, G)], dibuf.at[1 - p])
            for d in sd:
                d.wait()
            return carry

        lax.fori_loop(0, ng, group, 0)
        plsc.subcore_barrier()
        pltpu.sync_copy(acc.at[pl.ds(r0, rows_per_tile)],
                        out_hbm.at[pl.ds(c * n_dst + r0, rows_per_tile)])

    return phase


BE = 50
BV = 40
GE = 2
GV = 2
_phase_e = _make_sc_phase(NODE_PAD, EDGE_PAD, BE, GE)
_phase_v = _make_sc_phase(EDGE_PAD, NODE_PAD, BV, GV)


def kernel(input, V, E, weight, bias):
    x = jnp.pad(input.astype(jnp.float32), ((0, NODE_PAD - N_NODE), (0, 0)))
    v32 = V.astype(jnp.int32)
    e32 = E.astype(jnp.int32)
    wp = jnp.pad(weight.astype(jnp.float32), ((0, 0), (0, W - D)))
    z = jnp.zeros((NODE_PAD, W), jnp.float32)
    xp = _mm(x, wp)
    ep = _phase_e(xp, v32.reshape(NNZ // BE, BE), e32.reshape(NNZ // BE, BE),
                  z)
    xe = _merge(ep)
    vp = _phase_v(xe, e32.reshape(NNZ // BV, BV), v32.reshape(NNZ // BV, BV),
                  z)
    out = _final(vp, bias.reshape(1, D).astype(jnp.float32))
    return out[:N_NODE]


# speedup vs baseline: 1.3723x; 1.3723x over previous
import functools

import jax
import jax.numpy as jnp
from jax import lax
from jax.experimental import pallas as pl
from jax.experimental.pallas import tpu as pltpu
from jax.experimental.pallas import tpu_sc as plsc

N_NODE = 10000
N_EDGE = 2000
NNZ = 320000
D = 128
W = 144
ONES_COL = 128
EDGE_PAD = 2048
NODE_PAD = 10240
BLK = 1000
FBLK = 1024


def _mm_body(x_ref, w_ref, o_ref):
    acc = jnp.dot(x_ref[...], w_ref[...], preferred_element_type=jnp.float32)
    col = lax.broadcasted_iota(jnp.int32, (BLK, W), 1)
    o_ref[...] = acc + jnp.where(col == ONES_COL, 1.0, 0.0)


def _mm(x, wp):
    return pl.pallas_call(
        _mm_body,
        grid=(N_NODE // BLK,),
        in_specs=[
            pl.BlockSpec((BLK, D), lambda i: (i, 0)),
            pl.BlockSpec((D, W), lambda i: (0, 0)),
        ],
        out_specs=pl.BlockSpec((BLK, W), lambda i: (i, 0)),
        out_shape=jax.ShapeDtypeStruct((N_NODE, W), jnp.float32),
    )(x, wp)


def _merge_body(a_ref, b_ref, o_ref):
    t = a_ref[...] + b_ref[...]
    cnt = jnp.maximum(t[:, ONES_COL:ONES_COL + 1], 1.0)
    o_ref[...] = t / cnt


def _merge(ep):
    return pl.pallas_call(
        _merge_body,
        grid=(1,),
        in_specs=[
            pl.BlockSpec((EDGE_PAD, W), lambda i: (0, 0)),
            pl.BlockSpec((EDGE_PAD, W), lambda i: (1, 0)),
        ],
        out_specs=pl.BlockSpec((EDGE_PAD, W), lambda i: (0, 0)),
        out_shape=jax.ShapeDtypeStruct((EDGE_PAD, W), jnp.float32),
    )(ep, ep)


def _final_body(a_ref, b_ref, bias_ref, o_ref):
    t = a_ref[...] + b_ref[...]
    cnt = jnp.maximum(t[:, ONES_COL:ONES_COL + 1], 1.0)
    o_ref[...] = t[:, :D] / cnt + bias_ref[...]


def _final(vp, bias2d):
    nb = NODE_PAD // FBLK
    return pl.pallas_call(
        _final_body,
        grid=(nb,),
        in_specs=[
            pl.BlockSpec((FBLK, W), lambda i: (i, 0)),
            pl.BlockSpec((FBLK, W), lambda i: (i + nb, 0)),
            pl.BlockSpec((1, D), lambda i: (0, 0)),
        ],
        out_specs=pl.BlockSpec((FBLK, D), lambda i: (i, 0)),
        out_shape=jax.ShapeDtypeStruct((NODE_PAD, D), jnp.float32),
    )(vp, vp, bias2d)


def _make_sc_phase(n_src, n_dst, B, G):
    rows_per_tile = n_dst // 16
    chunk = NNZ // 32
    nb = chunk // B
    ng = nb // G
    assert nb % G == 0
    mesh = plsc.VectorSubcoreMesh(core_axis_name="c", subcore_axis_name="s")

    @functools.partial(
        pl.kernel,
        mesh=mesh,
        compiler_params=pltpu.CompilerParams(use_tc_tiling_on_sc=False),
        out_type=jax.ShapeDtypeStruct((2 * n_dst, W), jnp.float32),
        scratch_types=[
            pltpu.VMEM((G, B), jnp.int32),
            pltpu.VMEM((2, G, B), jnp.int32),
            pltpu.VMEM_SHARED((n_dst, W), jnp.float32),
        ]
        + [pltpu.VMEM((B, W), jnp.float32) for _ in range(G)]
        + [pltpu.SemaphoreType.DMA for _ in range(2 * G)],
    )
    def phase(src_hbm, gidx_hbm, didx_hbm, z_hbm, out_hbm,
              gibuf, dibuf, acc, *rest):
        rows = rest[:G]
        gsem = rest[G:2 * G]
        ssem = rest[2 * G:]
        c = lax.axis_index("c")
        s = lax.axis_index("s")
        r0 = s * rows_per_tile
        pltpu.sync_copy(z_hbm.at[pl.ds(r0, rows_per_tile)],
                        acc.at[pl.ds(r0, rows_per_tile)])
        tb = (c * 16 + s) * nb
        pltpu.sync_copy(gidx_hbm.at[pl.ds(tb, G)], gibuf)
        pltpu.sync_copy(didx_hbm.at[pl.ds(tb, G)], dibuf.at[0])
        plsc.subcore_barrier()

        def group(g, carry):
            p = lax.rem(g, 2)
            gd = [pltpu.async_copy(src_hbm.at[gibuf.at[b]],
                                   rows[b], gsem[b]) for b in range(G)]
            sd = []
            for b in range(G):
                gd[b].wait()
                sd.append(pltpu.async_copy(rows[b], acc.at[dibuf.at[p, b]],
                                           ssem[b], add=True))
            nxt = tb + jnp.minimum((g + 1) * G, nb - G)
            pltpu.sync_copy(gidx_hbm.at[pl.ds(nxt, G)], gibuf)
            pltpu.sync_copy(didx_hbm.at[pl.ds(nxt, G)], dibuf.at[1 - p])
            for d in sd:
                d.wait()
            return carry

        lax.fori_loop(0, ng, group, 0)
        plsc.subcore_barrier()
        pltpu.sync_copy(acc.at[pl.ds(r0, rows_per_tile)],
                        out_hbm.at[pl.ds(c * n_dst + r0, rows_per_tile)])

    return phase


BE = 100
BV = 40
_phase_e = _make_sc_phase(N_NODE, EDGE_PAD, BE, 5)
_phase_v = _make_sc_phase(EDGE_PAD, NODE_PAD, BV, 5)


def kernel(input, V, E, weight, bias):
    x = input.astype(jnp.float32)
    v32 = V.astype(jnp.int32)
    e32 = E.astype(jnp.int32)
    wp = jnp.pad(weight.astype(jnp.float32), ((0, 0), (0, W - D)))
    z = jnp.zeros((NODE_PAD, W), jnp.float32)
    xp = _mm(x, wp)
    ep = _phase_e(xp, v32.reshape(NNZ // BE, BE), e32.reshape(NNZ // BE, BE),
                  z)
    xe = _merge(ep)
    vp = _phase_v(xe, e32.reshape(NNZ // BV, BV), v32.reshape(NNZ // BV, BV),
                  z)
    out = _final(vp, bias.reshape(1, D).astype(jnp.float32))
    return out[:N_NODE]


# speedup vs baseline: 1.4291x; 1.0414x over previous
import functools

import jax
import jax.numpy as jnp
from jax import lax
from jax.experimental import pallas as pl
from jax.experimental.pallas import tpu as pltpu
from jax.experimental.pallas import tpu_sc as plsc

N_NODE = 10000
N_EDGE = 2000
NNZ = 320000
D = 128
W = 144
ONES_COL = 128
EDGE_PAD = 2048
NODE_PAD = 10240
BLK = 1000
FBLK = 1024


def _mm_body(x_ref, w_ref, o_ref):
    acc = jnp.dot(x_ref[...], w_ref[...], preferred_element_type=jnp.float32)
    col = lax.broadcasted_iota(jnp.int32, (BLK, W), 1)
    o_ref[...] = acc + jnp.where(col == ONES_COL, 1.0, 0.0)


def _mm(x, wp):
    return pl.pallas_call(
        _mm_body,
        grid=(N_NODE // BLK,),
        in_specs=[
            pl.BlockSpec((BLK, D), lambda i: (i, 0)),
            pl.BlockSpec((D, W), lambda i: (0, 0)),
        ],
        out_specs=pl.BlockSpec((BLK, W), lambda i: (i, 0)),
        out_shape=jax.ShapeDtypeStruct((N_NODE, W), jnp.float32),
    )(x, wp)


def _merge_body(a_ref, b_ref, o_ref):
    t = a_ref[...] + b_ref[...]
    cnt = jnp.maximum(t[:, ONES_COL:ONES_COL + 1], 1.0)
    o_ref[...] = t / cnt


def _merge(ep):
    return pl.pallas_call(
        _merge_body,
        grid=(1,),
        in_specs=[
            pl.BlockSpec((EDGE_PAD, W), lambda i: (0, 0)),
            pl.BlockSpec((EDGE_PAD, W), lambda i: (1, 0)),
        ],
        out_specs=pl.BlockSpec((EDGE_PAD, W), lambda i: (0, 0)),
        out_shape=jax.ShapeDtypeStruct((EDGE_PAD, W), jnp.float32),
    )(ep, ep)


def _final_body(a_ref, b_ref, bias_ref, o_ref):
    t = a_ref[...] + b_ref[...]
    cnt = jnp.maximum(t[:, ONES_COL:ONES_COL + 1], 1.0)
    o_ref[...] = t[:, :D] / cnt + bias_ref[...]


def _final(vp, bias2d):
    nb = NODE_PAD // FBLK
    return pl.pallas_call(
        _final_body,
        grid=(nb,),
        in_specs=[
            pl.BlockSpec((FBLK, W), lambda i: (i, 0)),
            pl.BlockSpec((FBLK, W), lambda i: (i + nb, 0)),
            pl.BlockSpec((1, D), lambda i: (0, 0)),
        ],
        out_specs=pl.BlockSpec((FBLK, D), lambda i: (i, 0)),
        out_shape=jax.ShapeDtypeStruct((NODE_PAD, D), jnp.float32),
    )(vp, vp, bias2d)


def _make_sc_phase(n_src, n_dst, B, G):
    rows_per_tile = n_dst // 16
    chunk = NNZ // 32
    nb = chunk // B
    ng = nb // G
    assert nb % G == 0
    mesh = plsc.VectorSubcoreMesh(core_axis_name="c", subcore_axis_name="s")

    @functools.partial(
        pl.kernel,
        mesh=mesh,
        compiler_params=pltpu.CompilerParams(use_tc_tiling_on_sc=False),
        out_type=jax.ShapeDtypeStruct((2 * n_dst, W), jnp.float32),
        scratch_types=[
            pltpu.VMEM((G, B), jnp.int32),
            pltpu.VMEM((2, G, B), jnp.int32),
            pltpu.VMEM_SHARED((n_dst, W), jnp.float32),
        ]
        + [pltpu.VMEM((B, W), jnp.float32) for _ in range(G)]
        + [pltpu.SemaphoreType.DMA for _ in range(2 * G)],
    )
    def phase(src_hbm, gidx_hbm, didx_hbm, z_hbm, out_hbm,
              gibuf, dibuf, acc, *rest):
        rows = rest[:G]
        gsem = rest[G:2 * G]
        ssem = rest[2 * G:]
        c = lax.axis_index("c")
        s = lax.axis_index("s")
        r0 = s * rows_per_tile
        pltpu.sync_copy(z_hbm.at[pl.ds(r0, rows_per_tile)],
                        acc.at[pl.ds(r0, rows_per_tile)])
        tb = (c * 16 + s) * nb
        pltpu.sync_copy(gidx_hbm.at[pl.ds(tb, G)], gibuf)
        pltpu.sync_copy(didx_hbm.at[pl.ds(tb, G)], dibuf.at[0])
        plsc.subcore_barrier()

        def group(g, carry):
            p = lax.rem(g, 2)
            gd = [pltpu.async_copy(src_hbm.at[gibuf.at[b]],
                                   rows[b], gsem[b]) for b in range(G)]
            sd = []
            for b in range(G):
                gd[b].wait()
                sd.append(pltpu.async_copy(rows[b], acc.at[dibuf.at[p, b]],
                                           ssem[b], add=True))
            nxt = tb + jnp.minimum((g + 1) * G, nb - G)
            pltpu.sync_copy(gidx_hbm.at[pl.ds(nxt, G)], gibuf)
            pltpu.sync_copy(didx_hbm.at[pl.ds(nxt, G)], dibuf.at[1 - p])
            for d in sd:
                d.wait()
            return carry

        lax.fori_loop(0, ng, group, 0)
        plsc.subcore_barrier()
        pltpu.sync_copy(acc.at[pl.ds(r0, rows_per_tile)],
                        out_hbm.at[pl.ds(c * n_dst + r0, rows_per_tile)])

    return phase


BE = 80
BV = 50
_phase_e = _make_sc_phase(N_NODE, EDGE_PAD, BE, 5)
_phase_v = _make_sc_phase(EDGE_PAD, NODE_PAD, BV, 5)


def kernel(input, V, E, weight, bias):
    x = input.astype(jnp.float32)
    v32 = V.astype(jnp.int32)
    e32 = E.astype(jnp.int32)
    wp = jnp.pad(weight.astype(jnp.float32), ((0, 0), (0, W - D)))
    z = jnp.zeros((NODE_PAD, W), jnp.float32)
    xp = _mm(x, wp)
    ep = _phase_e(xp, v32.reshape(NNZ // BE, BE), e32.reshape(NNZ // BE, BE),
                  z)
    xe = _merge(ep)
    vp = _phase_v(xe, e32.reshape(NNZ // BV, BV), v32.reshape(NNZ // BV, BV),
                  z)
    out = _final(vp, bias.reshape(1, D).astype(jnp.float32))
    return out[:N_NODE]
